# baseline (speedup 1.0000x reference)
import functools

import jax
import jax.numpy as jnp
from jax import lax
from jax.experimental import pallas as pl
from jax.experimental.pallas import tpu as pltpu
from jax.experimental.pallas import tpu_sc as plsc

_B, _C, _K, _D = 4096, 20, 20, 64
_NC, _NS = 2, 16
_NW = _NC * _NS
_EPW = _B // _NW
_E = 16
_NCHUNK = _EPW // _E
_P = _E * _C
_GSUB = 4
_SUB = _P // _GSUB


def _sc_pred(ctx_idx, foc_idx, cemb, nemb):
    mesh = plsc.VectorSubcoreMesh(core_axis_name="c", subcore_axis_name="s")

    @functools.partial(
        pl.kernel,
        out_type=jax.ShapeDtypeStruct((_B * _K,), jnp.float32),
        mesh=mesh,
        scratch_types=[
            pltpu.VMEM((_P,), jnp.int32),
            pltpu.VMEM((_P,), jnp.int32),
            pltpu.VMEM((_P, _D), jnp.float32),
            pltpu.VMEM((_P, _D), jnp.float32),
            pltpu.VMEM((_E, _D), jnp.float32),
            pltpu.VMEM((_P,), jnp.float32),
            pltpu.SemaphoreType.DMA,
            pltpu.SemaphoreType.DMA,
        ],
        compiler_params=pltpu.CompilerParams(
            use_tc_tiling_on_sc=False, needs_layout_passes=False),
    )
    def k(ci_hbm, fi_hbm, ce_hbm, ne_hbm, pred_hbm,
          ci_v, fi_v, cr_v, tr_v, src_v, pr_v, sem1, sem2):
        wid = lax.axis_index("s") * _NC + lax.axis_index("c")
        lane = lax.iota(jnp.int32, 16)

        def chunk_body(c, carry):
            po = (wid * _EPW + c * _E) * _C
            pltpu.sync_copy(ci_hbm.at[pl.ds(po, _P)], ci_v)
            pltpu.sync_copy(fi_hbm.at[pl.ds(po, _P)], fi_v)
            copies = []
            for i in range(_GSUB):
                s = pl.ds(i * _SUB, _SUB)
                copies.append(
                    pltpu.async_copy(ce_hbm.at[ci_v.at[s]], cr_v.at[s], sem1))
                copies.append(
                    pltpu.async_copy(ne_hbm.at[fi_v.at[s]], tr_v.at[s], sem2))
            for cp in copies:
                cp.wait()

            def ebody(e, ecarry):
                base = e * _C
                for d4 in range(_D // 16):
                    sl = pl.ds(d4 * 16, 16)
                    acc = cr_v[base, sl]
                    for cc in range(1, _C):
                        acc = acc + cr_v[base + cc, sl]
                    src_v[e, sl] = acc
                return ecarry
            lax.fori_loop(0, _E, ebody, 0)

            def gbody(g, gcarry):
                row = g * 16 + lane
                b_loc = row // _K
                acc = jnp.zeros((16,), jnp.float32)
                for d in range(_D):
                    dsp = jnp.full((16,), d, jnp.int32)
                    s = plsc.load_gather(src_v, [b_loc, dsp])
                    t = plsc.load_gather(tr_v, [row, dsp])
                    acc = acc + s * t
                pr_v[pl.ds(g * 16, 16)] = acc
                return gcarry
            lax.fori_loop(0, _P // 16, gbody, 0)

            pltpu.sync_copy(pr_v, pred_hbm.at[pl.ds(po, _P)])
            return carry

        lax.fori_loop(0, _NCHUNK, chunk_body, 0)

    return k(ctx_idx, foc_idx, cemb, nemb)


_TW = 125
_TS = 8


def _tc_transpose_body(t_ref, o_ref):
    o_ref[...] = jnp.transpose(t_ref[...], (1, 2, 0))


def _tc_transpose(tab_t):
    V = tab_t.shape[1]
    n = V // _TW
    t3 = tab_t.reshape(_D, n, _TW)
    out = pl.pallas_call(
        _tc_transpose_body,
        grid=(n // _TS,),
        in_specs=[pl.BlockSpec((_D, _TS, _TW), lambda i: (0, i, 0))],
        out_specs=pl.BlockSpec((_TS, _TW, _D), lambda i: (i, 0, 0)),
        out_shape=jax.ShapeDtypeStruct((n, _TW, _D), jnp.float32),
    )(t3)
    return out.reshape(V, _D)


def _tc_loss_body(p_ref, w_ref, l_ref, o_ref):
    p = p_ref[...]
    w = w_ref[...]
    lbl = l_ref[...]
    bce = jnp.maximum(p, 0.0) - p * lbl + jnp.log1p(jnp.exp(-jnp.abs(p)))
    num = jnp.sum(w * bce, axis=1, keepdims=True)
    den = jnp.sum(w, axis=1, keepdims=True)
    o_ref[...] = jnp.sum(num / den, axis=0, keepdims=True) / p_ref.shape[0]


def kernel(input, focus_word, weight_mask, labels, context_emb, neg_emb):
    ci = input.reshape(-1)
    fi = focus_word.reshape(-1)
    ct = _tc_transpose(context_emb.T)
    nt = _tc_transpose(neg_emb.T)
    pred = _sc_pred(ci, fi, ct, nt)
    loss = pl.pallas_call(
        _tc_loss_body,
        out_shape=jax.ShapeDtypeStruct((1, 1), jnp.float32),
    )(pred.reshape(_B, _K), weight_mask, labels)
    return loss[0, 0]


# speedup vs baseline: 1.7652x; 1.7652x over previous
import functools

import jax
import jax.numpy as jnp
from jax import lax
from jax.experimental import pallas as pl
from jax.experimental.pallas import tpu as pltpu
from jax.experimental.pallas import tpu_sc as plsc

_B, _C, _K, _D = 4096, 20, 20, 64
_W = _D // 2
_NC, _NS = 2, 16
_NW = _NC * _NS
_EPW = _B // _NW
_E = 16
_NCHUNK = _EPW // _E
_P = _E * _C
_GSUB = 4
_SUB = _P // _GSUB


def _pack_bf16_pairs(tab):
    lo = lax.bitcast_convert_type(
        tab[:, 0::2].astype(jnp.bfloat16), jnp.uint16).astype(jnp.uint32)
    hi = lax.bitcast_convert_type(
        tab[:, 1::2].astype(jnp.bfloat16), jnp.uint16).astype(jnp.uint32)
    return lax.bitcast_convert_type(lo | (hi << 16), jnp.int32)


def _sc_pred(ctx_idx, foc_idx, cemb, nemb32):
    mesh = plsc.VectorSubcoreMesh(core_axis_name="c", subcore_axis_name="s")

    @functools.partial(
        pl.kernel,
        out_type=jax.ShapeDtypeStruct((_B * _K,), jnp.float32),
        mesh=mesh,
        scratch_types=[
            pltpu.VMEM((_P,), jnp.int32),
            pltpu.VMEM((_P,), jnp.int32),
            pltpu.VMEM((_P, _W), jnp.int32),
            pltpu.VMEM((_P, _W), jnp.int32),
            pltpu.VMEM((_E, _W), jnp.float32),
            pltpu.VMEM((_E, _W), jnp.float32),
            pltpu.VMEM((_P,), jnp.float32),
            pltpu.SemaphoreType.DMA,
            pltpu.SemaphoreType.DMA,
        ],
        compiler_params=pltpu.CompilerParams(
            use_tc_tiling_on_sc=False, needs_layout_passes=False),
    )
    def k(ci_hbm, fi_hbm, ce_hbm, ne_hbm, pred_hbm,
          ci_v, fi_v, cr_v, tr_v, se_v, so_v, pr_v, sem1, sem2):
        wid = lax.axis_index("s") * _NC + lax.axis_index("c")
        lane = lax.iota(jnp.int32, 16)
        himask = jnp.full((16,), -65536, jnp.int32)

        def chunk_body(c, carry):
            po = (wid * _EPW + c * _E) * _C
            pltpu.sync_copy(ci_hbm.at[pl.ds(po, _P)], ci_v)
            pltpu.sync_copy(fi_hbm.at[pl.ds(po, _P)], fi_v)
            copies = []
            for i in range(_GSUB):
                s = pl.ds(i * _SUB, _SUB)
                copies.append(
                    pltpu.async_copy(ce_hbm.at[ci_v.at[s]], cr_v.at[s], sem1))
                copies.append(
                    pltpu.async_copy(ne_hbm.at[fi_v.at[s]], tr_v.at[s], sem2))
            for cp in copies:
                cp.wait()

            def ebody(e, ecarry):
                base = e * _C
                for h in range(_W // 16):
                    sl = pl.ds(h * 16, 16)
                    w0 = cr_v[base, sl]
                    acc_e = plsc.bitcast(w0 << 16, jnp.float32)
                    acc_o = plsc.bitcast(w0 & himask, jnp.float32)
                    for cc in range(1, _C):
                        w = cr_v[base + cc, sl]
                        acc_e = acc_e + plsc.bitcast(w << 16, jnp.float32)
                        acc_o = acc_o + plsc.bitcast(w & himask, jnp.float32)
                    se_v[e, sl] = acc_e
                    so_v[e, sl] = acc_o
                return ecarry
            lax.fori_loop(0, _E, ebody, 0)

            def gbody(g, gcarry):
                row = g * 16 + lane
                b_loc = row // _K
                acc = jnp.zeros((16,), jnp.float32)
                for j in range(_W):
                    jsp = jnp.full((16,), j, jnp.int32)
                    wt = plsc.load_gather(tr_v, [row, jsp])
                    t_lo = plsc.bitcast(wt << 16, jnp.float32)
                    t_hi = plsc.bitcast(wt & himask, jnp.float32)
                    s_e = plsc.load_gather(se_v, [b_loc, jsp])
                    s_o = plsc.load_gather(so_v, [b_loc, jsp])
                    acc = acc + s_e * t_lo + s_o * t_hi
                pr_v[pl.ds(g * 16, 16)] = acc
                return gcarry
            lax.fori_loop(0, _P // 16, gbody, 0)

            pltpu.sync_copy(pr_v, pred_hbm.at[pl.ds(po, _P)])
            return carry

        lax.fori_loop(0, _NCHUNK, chunk_body, 0)

    return k(ctx_idx, foc_idx, cemb, nemb32)


def _tc_loss_body(p_ref, w_ref, l_ref, o_ref):
    p = p_ref[...]
    w = w_ref[...]
    lbl = l_ref[...]
    bce = jnp.maximum(p, 0.0) - p * lbl + jnp.log1p(jnp.exp(-jnp.abs(p)))
    num = jnp.sum(w * bce, axis=1, keepdims=True)
    den = jnp.sum(w, axis=1, keepdims=True)
    o_ref[...] = jnp.sum(num / den, axis=0, keepdims=True) / p_ref.shape[0]


def kernel(input, focus_word, weight_mask, labels, context_emb, neg_emb):
    ci = input.reshape(-1)
    fi = focus_word.reshape(-1)
    cb32 = _pack_bf16_pairs(context_emb)
    nb32 = _pack_bf16_pairs(neg_emb)
    pred = _sc_pred(ci, fi, cb32, nb32)
    loss = pl.pallas_call(
        _tc_loss_body,
        out_shape=jax.ShapeDtypeStruct((1, 1), jnp.float32),
    )(pred.reshape(_B, _K), weight_mask, labels)
    return loss[0, 0]


# speedup vs baseline: 5.0533x; 2.8628x over previous
import functools

import jax
import jax.numpy as jnp
from jax import lax
from jax.experimental import pallas as pl
from jax.experimental.pallas import tpu as pltpu
from jax.experimental.pallas import tpu_sc as plsc

_B, _C, _K, _D = 4096, 20, 20, 64
_W = _D // 2
_NC, _NS = 2, 16
_NW = _NC * _NS
_EPW = _B // _NW
_E = 16
_NCHUNK = _EPW // _E
_P = _E * _C
_GSUB = 4
_SUB = _P // _GSUB


def _pack_bf16_pairs(tab):
    lo = lax.bitcast_convert_type(
        tab[:, 0::2].astype(jnp.bfloat16), jnp.uint16).astype(jnp.uint32)
    hi = lax.bitcast_convert_type(
        tab[:, 1::2].astype(jnp.bfloat16), jnp.uint16).astype(jnp.uint32)
    return lax.bitcast_convert_type(lo | (hi << 16), jnp.int32)


def _sc_pred(ctx_idx, foc_idx, cemb, nemb32):
    mesh = plsc.VectorSubcoreMesh(core_axis_name="c", subcore_axis_name="s")

    @functools.partial(
        pl.kernel,
        out_type=jax.ShapeDtypeStruct((_B * _K,), jnp.float32),
        mesh=mesh,
        scratch_types=[
            pltpu.VMEM((_P,), jnp.int32),
            pltpu.VMEM((_P,), jnp.int32),
            pltpu.VMEM((_P, _D), jnp.float32),
            pltpu.VMEM((_P, _D), jnp.bfloat16),
            pltpu.VMEM((_P, _W), jnp.int32),
            pltpu.VMEM((_E, _D), jnp.float32),
            pltpu.VMEM((_P,), jnp.float32),
            pltpu.SemaphoreType.DMA,
            pltpu.SemaphoreType.DMA,
        ],
        compiler_params=pltpu.CompilerParams(
            use_tc_tiling_on_sc=False, needs_layout_passes=False),
    )
    def k(ci_hbm, fi_hbm, ce_hbm, ne_hbm, pred_hbm,
          ci_v, fi_v, cr_v, tb_v, tr_v, src_v, pr_v, sem1, sem2):
        wid = lax.axis_index("s") * _NC + lax.axis_index("c")
        lane = lax.iota(jnp.int32, 16)
        himask = jnp.full((16,), -65536, jnp.int32)

        def chunk_body(c, carry):
            po = (wid * _EPW + c * _E) * _C
            pltpu.sync_copy(ci_hbm.at[pl.ds(po, _P)], ci_v)
            pltpu.sync_copy(fi_hbm.at[pl.ds(po, _P)], fi_v)
            copies = []
            for i in range(_GSUB):
                s = pl.ds(i * _SUB, _SUB)
                copies.append(
                    pltpu.async_copy(ce_hbm.at[ci_v.at[s]], cr_v.at[s], sem1))
                copies.append(
                    pltpu.async_copy(ne_hbm.at[fi_v.at[s]], tb_v.at[s], sem2))
            for cp in copies:
                cp.wait()

            def cbody(r, ccarry):
                for h in range(_D // 32):
                    w = plsc.bitcast(tb_v[r, pl.ds(h * 32, 32)], jnp.int32)
                    tr_v[r, pl.ds(h * 16, 16)] = w
                return ccarry
            lax.fori_loop(0, _P, cbody, 0)

            def ebody(e, ecarry):
                base = e * _C
                for d4 in range(_D // 16):
                    sl = pl.ds(d4 * 16, 16)
                    acc = cr_v[base, sl]
                    for cc in range(1, _C):
                        acc = acc + cr_v[base + cc, sl]
                    src_v[e, sl] = acc
                return ecarry
            lax.fori_loop(0, _E, ebody, 0)

            def gbody(g, gcarry):
                row = g * 16 + lane
                b_loc = row // _K
                acc = jnp.zeros((16,), jnp.float32)
                for j in range(_W):
                    jsp = jnp.full((16,), j, jnp.int32)
                    wt = plsc.load_gather(tr_v, [row, jsp])
                    t_lo = plsc.bitcast(wt << 16, jnp.float32)
                    t_hi = plsc.bitcast(wt & himask, jnp.float32)
                    s_e = plsc.load_gather(
                        src_v, [b_loc, jnp.full((16,), 2 * j, jnp.int32)])
                    s_o = plsc.load_gather(
                        src_v, [b_loc, jnp.full((16,), 2 * j + 1, jnp.int32)])
                    acc = acc + s_e * t_lo + s_o * t_hi
                pr_v[pl.ds(g * 16, 16)] = acc
                return gcarry
            lax.fori_loop(0, _P // 16, gbody, 0)

            pltpu.sync_copy(pr_v, pred_hbm.at[pl.ds(po, _P)])
            return carry

        lax.fori_loop(0, _NCHUNK, chunk_body, 0)

    return k(ctx_idx, foc_idx, cemb, nemb32)


def _tc_loss_body(p_ref, w_ref, l_ref, o_ref):
    p = p_ref[...]
    w = w_ref[...]
    lbl = l_ref[...]
    bce = jnp.maximum(p, 0.0) - p * lbl + jnp.log1p(jnp.exp(-jnp.abs(p)))
    num = jnp.sum(w * bce, axis=1, keepdims=True)
    den = jnp.sum(w, axis=1, keepdims=True)
    o_ref[...] = jnp.sum(num / den, axis=0, keepdims=True) / p_ref.shape[0]


def kernel(input, focus_word, weight_mask, labels, context_emb, neg_emb):
    ci = input.reshape(-1)
    fi = focus_word.reshape(-1)
    pred = _sc_pred(ci, fi, context_emb, neg_emb.astype(jnp.bfloat16))
    loss = pl.pallas_call(
        _tc_loss_body,
        out_shape=jax.ShapeDtypeStruct((1, 1), jnp.float32),
    )(pred.reshape(_B, _K), weight_mask, labels)
    return loss[0, 0]


# speedup vs baseline: 5.4765x; 1.0837x over previous
import functools

import jax
import jax.numpy as jnp
from jax import lax
from jax.experimental import pallas as pl
from jax.experimental.pallas import tpu as pltpu
from jax.experimental.pallas import tpu_sc as plsc

_B, _C, _K, _D = 4096, 20, 20, 64
_V2 = 500000
_NC, _NS = 2, 16
_NW = _NC * _NS
_EPW = _B // _NW
_E = 16
_NCHUNK = _EPW // _E
_P = _E * _C
_GSUB = 4
_SUB = _P // _GSUB


def _sc_pred(ctx_idx, foc_idx, cemb2, nemb2):
    mesh = plsc.VectorSubcoreMesh(core_axis_name="c", subcore_axis_name="s")

    @functools.partial(
        pl.kernel,
        out_type=jax.ShapeDtypeStruct((_B * _K,), jnp.float32),
        mesh=mesh,
        scratch_types=[
            pltpu.VMEM((_P,), jnp.int32),
            pltpu.VMEM((_P,), jnp.int32),
            pltpu.VMEM((_P,), jnp.int32),
            pltpu.VMEM((_P,), jnp.int32),
            pltpu.VMEM((_P,), jnp.int32),
            pltpu.VMEM((_P,), jnp.int32),
            pltpu.VMEM((_P, 2 * _D), jnp.float32),
            pltpu.VMEM((_P, 2 * _D), jnp.float32),
            pltpu.VMEM((_E, _D), jnp.float32),
            pltpu.VMEM((_P,), jnp.float32),
            pltpu.SemaphoreType.DMA,
            pltpu.SemaphoreType.DMA,
        ],
        compiler_params=pltpu.CompilerParams(
            use_tc_tiling_on_sc=False, needs_layout_passes=False),
    )
    def k(ci_hbm, fi_hbm, ce_hbm, ne_hbm, pred_hbm,
          ci_v, fi_v, cu_v, fu_v, ch_v, fh_v, cr_v, tr_v, src_v, pr_v,
          sem1, sem2):
        wid = lax.axis_index("s") * _NC + lax.axis_index("c")
        lane = lax.iota(jnp.int32, 16)

        def chunk_body(c, carry):
            po = (wid * _EPW + c * _E) * _C
            pltpu.sync_copy(ci_hbm.at[pl.ds(po, _P)], ci_v)
            pltpu.sync_copy(fi_hbm.at[pl.ds(po, _P)], fi_v)

            def ibody(i, icarry):
                sl = pl.ds(i * 16, 16)
                cw = ci_v[sl]
                cu_v[sl] = cw >> 1
                ch_v[sl] = (cw & 1) << 6
                fw = fi_v[sl]
                fu_v[sl] = fw >> 1
                fh_v[sl] = (fw & 1) << 6
                return icarry
            lax.fori_loop(0, _P // 16, ibody, 0)

            copies = []
            for i in range(_GSUB):
                s = pl.ds(i * _SUB, _SUB)
                copies.append(
                    pltpu.async_copy(ce_hbm.at[cu_v.at[s]], cr_v.at[s], sem1))
                copies.append(
                    pltpu.async_copy(ne_hbm.at[fu_v.at[s]], tr_v.at[s], sem2))
            for cp in copies:
                cp.wait()

            def ebody(e, ecarry):
                base = e * _C
                ha = ch_v[pl.ds(base, 16)]
                hb = ch_v[pl.ds(base + 4, 16)]
                for d4 in range(_D // 16):
                    h0 = ha[0]
                    acc = cr_v[base, pl.ds(h0 + d4 * 16, 16)]
                    for cc in range(1, _C):
                        h = ha[cc] if cc < 16 else hb[cc - 4]
                        acc = acc + cr_v[base + cc, pl.ds(h + d4 * 16, 16)]
                    src_v[e, pl.ds(d4 * 16, 16)] = acc
                return ecarry
            lax.fori_loop(0, _E, ebody, 0)

            def gbody(g, gcarry):
                row = g * 16 + lane
                b_loc = row // _K
                th = plsc.load_gather(fh_v, [row])
                acc = jnp.zeros((16,), jnp.float32)
                for d in range(_D):
                    dsp = jnp.full((16,), d, jnp.int32)
                    s = plsc.load_gather(src_v, [b_loc, dsp])
                    t = plsc.load_gather(tr_v, [row, th + dsp])
                    acc = acc + s * t
                pr_v[pl.ds(g * 16, 16)] = acc
                return gcarry
            lax.fori_loop(0, _P // 16, gbody, 0)

            pltpu.sync_copy(pr_v, pred_hbm.at[pl.ds(po, _P)])
            return carry

        lax.fori_loop(0, _NCHUNK, chunk_body, 0)

    return k(ctx_idx, foc_idx, cemb2, nemb2)


def _tc_loss_body(p_ref, w_ref, l_ref, o_ref):
    p = p_ref[...]
    w = w_ref[...]
    lbl = l_ref[...]
    bce = jnp.maximum(p, 0.0) - p * lbl + jnp.log1p(jnp.exp(-jnp.abs(p)))
    num = jnp.sum(w * bce, axis=1, keepdims=True)
    den = jnp.sum(w, axis=1, keepdims=True)
    o_ref[...] = jnp.sum(num / den, axis=0, keepdims=True) / p_ref.shape[0]


def kernel(input, focus_word, weight_mask, labels, context_emb, neg_emb):
    ci = input.reshape(-1)
    fi = focus_word.reshape(-1)
    ct2 = context_emb.reshape(_V2, 2 * _D)
    nt2 = neg_emb.reshape(_V2, 2 * _D)
    pred = _sc_pred(ci, fi, ct2, nt2)
    loss = pl.pallas_call(
        _tc_loss_body,
        out_shape=jax.ShapeDtypeStruct((1, 1), jnp.float32),
    )(pred.reshape(_B, _K), weight_mask, labels)
    return loss[0, 0]
